# baseline (speedup 1.0000x reference)
import jax
import jax.numpy as jnp
from jax import lax
from jax.experimental import pallas as pl
from jax.experimental.pallas import tpu as pltpu
from jax.experimental.pallas import tpu_sc as plsc

_B, _P, _N, _S, _C = 8, 1024, 4096, 32, 64
_D = 3 + _C
_DP = 80
_L = 16
_NC, _NS = 2, 16
_NW = _NC * _NS
_PC = (_B * _P) // _NW
_QW = _P // _PC
_G = 4
_GR = _G * _S
_NCHUNK = _N // _L
_U = 8
_T2 = float.fromhex("0x1.47ae14p-5")
_INF = float("inf")
_BIGI = 2 ** 30


def _sc_body(cxt, xyzt, aug, npts, idxo,
             xv, yv, zv, ccx, ccy, ccz, cd, ci, idxg, gidx, rows, sem):
    wid = lax.axis_index("s") * _NC + lax.axis_index("c")
    b = wid // _QW
    cb = (wid % _QW) * _PC

    pltpu.sync_copy(xyzt.at[pl.ds((b * 3 + 0) * _N, _N)], xv)
    pltpu.sync_copy(xyzt.at[pl.ds((b * 3 + 1) * _N, _N)], yv)
    pltpu.sync_copy(xyzt.at[pl.ds((b * 3 + 2) * _N, _N)], zv)
    pltpu.sync_copy(cxt.at[pl.ds((b * 3 + 0) * _P + cb, _PC)], ccx)
    pltpu.sync_copy(cxt.at[pl.ds((b * 3 + 1) * _P + cb, _PC)], ccy)
    pltpu.sync_copy(cxt.at[pl.ds((b * 3 + 2) * _P + cb, _PC)], ccz)

    lane = lax.broadcasted_iota(jnp.int32, (_L,), 0)
    inf16 = jnp.full((_L,), _INF, jnp.float32)
    zero16i = jnp.zeros((_L,), jnp.int32)

    def centre(i, carry):
        isplat = jnp.full((_L,), i, jnp.int32)
        cx = plsc.load_gather(ccx, [isplat])
        cy = plsc.load_gather(ccy, [isplat])
        cz = plsc.load_gather(ccz, [isplat])

        def chunk(j, offv):
            for u in range(_U):
                base = (j * _U + u) * _L
                dx = xv[pl.ds(base, _L)] - cx
                dy = yv[pl.ds(base, _L)] - cy
                dz = zv[pl.ds(base, _L)] - cz
                d2 = dx * dx + dy * dy + dz * dz
                msk = d2 < _T2
                pos = offv + plsc.cumsum(msk.astype(jnp.int32)) - 1
                plsc.store_scatter(cd, [pos], d2, mask=msk)
                plsc.store_scatter(ci, [pos], lane + base, mask=msk)
                offv = offv + plsc.all_reduce_population_count(msk)
            return offv

        offv = lax.fori_loop(0, _NCHUNK // _U, chunk, zero16i)
        mcount = jnp.sum(jnp.where(lane == 0, offv, 0))

        def no_cand(_):
            def amin_chunk(j, st):
                rmin, ridx = st
                base = j * _L
                dx = xv[pl.ds(base, _L)] - cx
                dy = yv[pl.ds(base, _L)] - cy
                dz = zv[pl.ds(base, _L)] - cz
                d2 = dx * dx + dy * dy + dz * dz
                upd = d2 < rmin
                return jnp.where(upd, d2, rmin), jnp.where(upd, lane + base, ridx)

            rmin, ridx = lax.fori_loop(0, _NCHUNK, amin_chunk, (inf16, zero16i))
            mval = jnp.min(rmin)
            am = jnp.min(jnp.where(rmin == mval, ridx, jnp.int32(_BIGI)))
            cd[pl.ds(0, _L)] = jnp.where(lane == 0, jnp.float32(0.0), inf16)
            ci[pl.ds(0, _L)] = jnp.full((_L,), am, jnp.int32)
            return jnp.int32(1)

        def have_cand(_):
            cd[pl.ds(mcount, _L)] = inf16
            return mcount

        m = lax.cond(mcount == 0, no_cand, have_cand, 0)
        nc = (m + _L - 1) // _L
        gl = (i % _G) * _S

        def merge(c, st):
            k0, v0, k1, v1 = st
            dk, dv = plsc.sort_key_val(cd[pl.ds(c * _L, _L)],
                                       ci[pl.ds(c * _L, _L)],
                                       descending=True)
            m1 = k1 <= dk
            tk = jnp.where(m1, k1, dk)
            tv = jnp.where(m1, v1, dv)
            tk, tv = plsc.sort_key_val(tk, tv, descending=True)
            m2 = k0 <= tk
            lok = jnp.where(m2, k0, tk)
            lov = jnp.where(m2, v0, tv)
            hik = jnp.where(m2, tk, k0)
            hiv = jnp.where(m2, tv, v0)
            k0n, v0n = plsc.sort_key_val(lok, lov)
            k1n, v1n = plsc.sort_key_val(hik, hiv)
            return k0n, v0n, k1n, v1n

        k0, v0, k1, v1 = lax.fori_loop(0, nc, merge,
                                       (inf16, zero16i, inf16, zero16i))
        c0 = jnp.sum(jnp.where(lane == 0, v0, 0))
        c0v = jnp.full((_L,), c0, jnp.int32)
        idxg[pl.ds(gl, _L)] = jnp.where(k0 == _INF, c0v, v0)
        idxg[pl.ds(gl + _L, _L)] = jnp.where(k1 == _INF, c0v, v1)

        @pl.when(i % _G == _G - 1)
        def _():
            boff = jnp.full((_L,), b * _N, jnp.int32)
            for t in range(_GR // _L):
                gidx[pl.ds(t * _L, _L)] = idxg[pl.ds(t * _L, _L)] + boff
            rowbase = (b * _P + cb + (i - (_G - 1))) * _S
            pltpu.sync_copy(idxg, idxo.at[pl.ds(rowbase, _GR)])
            pltpu.async_copy(aug.at[gidx], rows, sem).wait()
            pltpu.sync_copy(rows, npts.at[pl.ds(rowbase, _GR)])

        return carry

    lax.fori_loop(0, _PC, centre, jnp.int32(0))


def kernel(new_xyz, xyz, points):
    cxt = jnp.transpose(new_xyz, (0, 2, 1)).reshape(-1)
    xyzt = jnp.transpose(xyz, (0, 2, 1)).reshape(-1)
    pad = jnp.zeros((_B, _N, _DP - _D), jnp.float32)
    aug = jnp.concatenate([xyz, points, pad], axis=-1).reshape(_B * _N, _DP)

    mesh = plsc.VectorSubcoreMesh(core_axis_name="c", subcore_axis_name="s",
                                  num_cores=_NC, num_subcores=_NS)
    out_type = (
        jax.ShapeDtypeStruct((_B * _P * _S, _DP), jnp.float32),
        jax.ShapeDtypeStruct((_B * _P * _S,), jnp.int32),
    )
    scratch = [
        pltpu.VMEM((_N,), jnp.float32),
        pltpu.VMEM((_N,), jnp.float32),
        pltpu.VMEM((_N,), jnp.float32),
        pltpu.VMEM((_PC,), jnp.float32),
        pltpu.VMEM((_PC,), jnp.float32),
        pltpu.VMEM((_PC,), jnp.float32),
        pltpu.VMEM((_N + _L,), jnp.float32),
        pltpu.VMEM((_N + _L,), jnp.int32),
        pltpu.VMEM((_GR,), jnp.int32),
        pltpu.VMEM((_GR,), jnp.int32),
        pltpu.VMEM((_GR, _DP), jnp.float32),
        pltpu.SemaphoreType.DMA,
    ]
    f = pl.kernel(_sc_body, out_type=out_type, mesh=mesh, scratch_types=scratch,
                  compiler_params=pltpu.CompilerParams(
                      needs_layout_passes=False, use_tc_tiling_on_sc=False))
    nppad, idxf = f(cxt, xyzt, aug)
    nppad = nppad.reshape(_B, _P, _S, _DP)
    return (nppad[..., :_D], idxf.reshape(_B, _P, _S), nppad[..., :3])


# speedup vs baseline: 1.8743x; 1.8743x over previous
import jax
import jax.numpy as jnp
from jax import lax
from jax.experimental import pallas as pl
from jax.experimental.pallas import tpu as pltpu
from jax.experimental.pallas import tpu_sc as plsc

_B, _P, _N, _S, _C = 8, 1024, 4096, 32, 64
_D = 3 + _C
_DP = 80
_L = 16
_NC, _NS = 2, 16
_NW = _NC * _NS
_PC = (_B * _P) // _NW
_QW = _P // _PC
_G = 4
_GR = _G * _S
_NCHUNK = _N // _L
_NBX = 8
_NB = _NBX * _NBX
_NBP = 80
_R = 0.2
_SUB1 = float.fromhex("0x1.fffffep-1")
_T2 = float.fromhex("0x1.47ae14p-5")
_INF = float("inf")
_BIGI = 2 ** 30


def _sc_body(cxt, xyzt, aug, npts, idxo,
             xv, yv, zv, ccx, ccy, ccz, cd, ci, idxg, gidx, rows,
             bkt, xs, ys, zs, pidx, offs, cur, sem):
    wid = lax.axis_index("s") * _NC + lax.axis_index("c")
    b = wid // _QW
    cb = (wid % _QW) * _PC

    pltpu.sync_copy(xyzt.at[pl.ds((b * 3 + 0) * _N, _N)], xv)
    pltpu.sync_copy(xyzt.at[pl.ds((b * 3 + 1) * _N, _N)], yv)
    pltpu.sync_copy(xyzt.at[pl.ds((b * 3 + 2) * _N, _N)], zv)
    pltpu.sync_copy(cxt.at[pl.ds((b * 3 + 0) * _P + cb, _PC)], ccx)
    pltpu.sync_copy(cxt.at[pl.ds((b * 3 + 1) * _P + cb, _PC)], ccy)
    pltpu.sync_copy(cxt.at[pl.ds((b * 3 + 2) * _P + cb, _PC)], ccz)

    lane = lax.broadcasted_iota(jnp.int32, (_L,), 0)
    inf16 = jnp.full((_L,), _INF, jnp.float32)
    zero16i = jnp.zeros((_L,), jnp.int32)
    ones16 = jnp.full((_L,), 1, jnp.int32)

    for t in range(_NBP // _L):
        cur[pl.ds(t * _L, _L)] = zero16i

    def count_chunk(j, _):
        base = j * _L
        xb = jnp.clip((xv[pl.ds(base, _L)] * float(_NBX)).astype(jnp.int32),
                      0, _NBX - 1)
        yb = jnp.clip((yv[pl.ds(base, _L)] * float(_NBX)).astype(jnp.int32),
                      0, _NBX - 1)
        bk = xb * _NBX + yb
        bkt[pl.ds(base, _L)] = bk
        plsc.addupdate_scatter(cur, [bk], ones16)
        return 0

    lax.fori_loop(0, _NCHUNK, count_chunk, 0)

    def prefix(t, car):
        c = cur[pl.ds(t * _L, _L)]
        excl = plsc.cumsum(c) - c + jnp.full((_L,), car, jnp.int32)
        offs[pl.ds(t * _L, _L)] = excl
        cur[pl.ds(t * _L, _L)] = excl
        return car + jnp.sum(c)

    lax.fori_loop(0, _NB // _L, prefix, jnp.int32(0))
    offs[pl.ds(_NB, _L)] = jnp.full((_L,), jnp.int32(_N))

    def place_chunk(j, _):
        base = j * _L
        bk = bkt[pl.ds(base, _L)]
        rank = zero16i
        for d in range(1, _L):
            nb = plsc.load_gather(bkt, [jnp.maximum(lane - d, 0) + base])
            rank = rank + jnp.where((lane >= d) & (bk == nb), 1, 0)
        pos = plsc.load_gather(cur, [bk]) + rank
        plsc.addupdate_scatter(cur, [bk], ones16)
        plsc.store_scatter(xs, [pos], xv[pl.ds(base, _L)])
        plsc.store_scatter(ys, [pos], yv[pl.ds(base, _L)])
        plsc.store_scatter(zs, [pos], zv[pl.ds(base, _L)])
        plsc.store_scatter(pidx, [pos], lane + base)
        return 0

    lax.fori_loop(0, _NCHUNK, place_chunk, 0)

    def centre(i, carry):
        isplat = jnp.full((_L,), i, jnp.int32)
        cx = plsc.load_gather(ccx, [isplat])
        cy = plsc.load_gather(ccy, [isplat])
        cz = plsc.load_gather(ccz, [isplat])

        xb0v = (jnp.maximum(cx - _R, 0.0) * float(_NBX)).astype(jnp.int32)
        xb1v = (jnp.minimum(cx + _R, _SUB1) * float(_NBX)).astype(jnp.int32)
        yb0v = (jnp.maximum(cy - _R, 0.0) * float(_NBX)).astype(jnp.int32)
        yb1v = (jnp.minimum(cy + _R, _SUB1) * float(_NBX)).astype(jnp.int32)
        xb0 = jnp.sum(jnp.where(lane == 0, xb0v, 0))
        xb1 = jnp.sum(jnp.where(lane == 0, xb1v, 0))
        yb0 = jnp.sum(jnp.where(lane == 0, yb0v, 0))
        yb1 = jnp.sum(jnp.where(lane == 0, yb1v, 0))

        def strip(xb, offv_s):
            s0 = jnp.sum(jnp.where(
                lane == 0,
                plsc.load_gather(offs, [jnp.full((_L,), xb * _NBX + yb0,
                                                 jnp.int32)]), 0))
            s1 = jnp.sum(jnp.where(
                lane == 0,
                plsc.load_gather(offs, [jnp.full((_L,), xb * _NBX + yb1 + 1,
                                                 jnp.int32)]), 0))

            def chunkq(j, offv):
                base = j * _L
                pi = lane + base
                dx = xs[pl.ds(base, _L)] - cx
                dy = ys[pl.ds(base, _L)] - cy
                dz = zs[pl.ds(base, _L)] - cz
                d2 = dx * dx + dy * dy + dz * dz
                msk = (d2 < _T2) & (pi >= s0) & (pi < s1)
                pos = offv + plsc.cumsum(msk.astype(jnp.int32)) - 1
                plsc.store_scatter(cd, [pos], d2, mask=msk)
                plsc.store_scatter(ci, [pos], pidx[pl.ds(base, _L)], mask=msk)
                return offv + plsc.all_reduce_population_count(msk)

            return lax.fori_loop(s0 // _L, (s1 + _L - 1) // _L, chunkq, offv_s)

        offv = lax.fori_loop(xb0, xb1 + 1, strip, zero16i)
        mcount = jnp.sum(jnp.where(lane == 0, offv, 0))

        def no_cand(_):
            def amin_chunk(j, st):
                rmin, ridx = st
                base = j * _L
                dx = xv[pl.ds(base, _L)] - cx
                dy = yv[pl.ds(base, _L)] - cy
                dz = zv[pl.ds(base, _L)] - cz
                d2 = dx * dx + dy * dy + dz * dz
                upd = d2 < rmin
                return jnp.where(upd, d2, rmin), jnp.where(upd, lane + base, ridx)

            rmin, ridx = lax.fori_loop(0, _NCHUNK, amin_chunk, (inf16, zero16i))
            mval = jnp.min(rmin)
            am = jnp.min(jnp.where(rmin == mval, ridx, jnp.int32(_BIGI)))
            cd[pl.ds(0, _L)] = jnp.where(lane == 0, jnp.float32(0.0), inf16)
            ci[pl.ds(0, _L)] = jnp.full((_L,), am, jnp.int32)
            return jnp.int32(1)

        def have_cand(_):
            cd[pl.ds(mcount, _L)] = inf16
            return mcount

        m = lax.cond(mcount == 0, no_cand, have_cand, 0)
        nc = (m + _L - 1) // _L
        gl = (i % _G) * _S

        def merge(c, st):
            k0, v0, k1, v1 = st
            dk, dv = plsc.sort_key_val(cd[pl.ds(c * _L, _L)],
                                       ci[pl.ds(c * _L, _L)],
                                       descending=True)
            m1 = k1 <= dk
            tk = jnp.where(m1, k1, dk)
            tv = jnp.where(m1, v1, dv)
            tk, tv = plsc.sort_key_val(tk, tv, descending=True)
            m2 = k0 <= tk
            lok = jnp.where(m2, k0, tk)
            lov = jnp.where(m2, v0, tv)
            hik = jnp.where(m2, tk, k0)
            hiv = jnp.where(m2, tv, v0)
            k0n, v0n = plsc.sort_key_val(lok, lov)
            k1n, v1n = plsc.sort_key_val(hik, hiv)
            return k0n, v0n, k1n, v1n

        k0, v0, k1, v1 = lax.fori_loop(0, nc, merge,
                                       (inf16, zero16i, inf16, zero16i))
        c0 = jnp.sum(jnp.where(lane == 0, v0, 0))
        c0v = jnp.full((_L,), c0, jnp.int32)
        idxg[pl.ds(gl, _L)] = jnp.where(k0 == _INF, c0v, v0)
        idxg[pl.ds(gl + _L, _L)] = jnp.where(k1 == _INF, c0v, v1)

        @pl.when(i % _G == _G - 1)
        def _():
            boff = jnp.full((_L,), b * _N, jnp.int32)
            for t in range(_GR // _L):
                gidx[pl.ds(t * _L, _L)] = idxg[pl.ds(t * _L, _L)] + boff
            rowbase = (b * _P + cb + (i - (_G - 1))) * _S
            pltpu.sync_copy(idxg, idxo.at[pl.ds(rowbase, _GR)])
            pltpu.async_copy(aug.at[gidx], rows, sem).wait()
            pltpu.sync_copy(rows, npts.at[pl.ds(rowbase, _GR)])

        return carry

    lax.fori_loop(0, _PC, centre, jnp.int32(0))


def kernel(new_xyz, xyz, points):
    cxt = jnp.transpose(new_xyz, (0, 2, 1)).reshape(-1)
    xyzt = jnp.transpose(xyz, (0, 2, 1)).reshape(-1)
    pad = jnp.zeros((_B, _N, _DP - _D), jnp.float32)
    aug = jnp.concatenate([xyz, points, pad], axis=-1).reshape(_B * _N, _DP)

    mesh = plsc.VectorSubcoreMesh(core_axis_name="c", subcore_axis_name="s",
                                  num_cores=_NC, num_subcores=_NS)
    out_type = (
        jax.ShapeDtypeStruct((_B * _P * _S, _DP), jnp.float32),
        jax.ShapeDtypeStruct((_B * _P * _S,), jnp.int32),
    )
    scratch = [
        pltpu.VMEM((_N,), jnp.float32),
        pltpu.VMEM((_N,), jnp.float32),
        pltpu.VMEM((_N,), jnp.float32),
        pltpu.VMEM((_PC,), jnp.float32),
        pltpu.VMEM((_PC,), jnp.float32),
        pltpu.VMEM((_PC,), jnp.float32),
        pltpu.VMEM((_N + _L,), jnp.float32),
        pltpu.VMEM((_N + _L,), jnp.int32),
        pltpu.VMEM((_GR,), jnp.int32),
        pltpu.VMEM((_GR,), jnp.int32),
        pltpu.VMEM((_GR, _DP), jnp.float32),
        pltpu.VMEM((_N,), jnp.int32),
        pltpu.VMEM((_N,), jnp.float32),
        pltpu.VMEM((_N,), jnp.float32),
        pltpu.VMEM((_N,), jnp.float32),
        pltpu.VMEM((_N,), jnp.int32),
        pltpu.VMEM((_NBP,), jnp.int32),
        pltpu.VMEM((_NBP,), jnp.int32),
        pltpu.SemaphoreType.DMA,
    ]
    f = pl.kernel(_sc_body, out_type=out_type, mesh=mesh, scratch_types=scratch,
                  compiler_params=pltpu.CompilerParams(
                      needs_layout_passes=False, use_tc_tiling_on_sc=False))
    nppad, idxf = f(cxt, xyzt, aug)
    nppad = nppad.reshape(_B, _P, _S, _DP)
    return (nppad[..., :_D], idxf.reshape(_B, _P, _S), nppad[..., :3])


# speedup vs baseline: 2.1173x; 1.1297x over previous
import jax
import jax.numpy as jnp
from jax import lax
from jax.experimental import pallas as pl
from jax.experimental.pallas import tpu as pltpu
from jax.experimental.pallas import tpu_sc as plsc

_B, _P, _N, _S, _C = 8, 1024, 4096, 32, 64
_D = 3 + _C
_DP = 80
_L = 16
_NC, _NS = 2, 16
_NW = _NC * _NS
_PC = (_B * _P) // _NW
_QW = _P // _PC
_G = 4
_GR = _G * _S
_NCHUNK = _N // _L
_NBX = 8
_NB = _NBX * _NBX
_NBP = 80
_R = 0.2
_SUB1 = float.fromhex("0x1.fffffep-1")
_T2 = float.fromhex("0x1.47ae14p-5")
_INF = float("inf")
_BIGI = 2 ** 30


def _sc_body(cxt, xyzt, aug, npts, idxo,
             xv, yv, zv, ccx, ccy, ccz, cd, ci,
             idxg0, idxg1, gidx0, gidx1, rows0, rows1,
             bkt, xs, ys, zs, pidx, offs, cur,
             gsem0, gsem1, wnp0, wnp1, wix0, wix1):
    wid = lax.axis_index("s") * _NC + lax.axis_index("c")
    b = wid // _QW
    cb = (wid % _QW) * _PC

    pltpu.sync_copy(xyzt.at[pl.ds((b * 3 + 0) * _N, _N)], xv)
    pltpu.sync_copy(xyzt.at[pl.ds((b * 3 + 1) * _N, _N)], yv)
    pltpu.sync_copy(xyzt.at[pl.ds((b * 3 + 2) * _N, _N)], zv)
    pltpu.sync_copy(cxt.at[pl.ds((b * 3 + 0) * _P + cb, _PC)], ccx)
    pltpu.sync_copy(cxt.at[pl.ds((b * 3 + 1) * _P + cb, _PC)], ccy)
    pltpu.sync_copy(cxt.at[pl.ds((b * 3 + 2) * _P + cb, _PC)], ccz)

    lane = lax.broadcasted_iota(jnp.int32, (_L,), 0)
    inf16 = jnp.full((_L,), _INF, jnp.float32)
    zero16i = jnp.zeros((_L,), jnp.int32)
    ones16 = jnp.full((_L,), 1, jnp.int32)

    for t in range(_NBP // _L):
        cur[pl.ds(t * _L, _L)] = zero16i

    def count_chunk(j, _):
        base = j * _L
        xb = jnp.clip((xv[pl.ds(base, _L)] * float(_NBX)).astype(jnp.int32),
                      0, _NBX - 1)
        yb = jnp.clip((yv[pl.ds(base, _L)] * float(_NBX)).astype(jnp.int32),
                      0, _NBX - 1)
        bk = xb * _NBX + yb
        bkt[pl.ds(base, _L)] = bk
        plsc.addupdate_scatter(cur, [bk], ones16)
        return 0

    lax.fori_loop(0, _NCHUNK, count_chunk, 0)

    def prefix(t, car):
        c = cur[pl.ds(t * _L, _L)]
        excl = plsc.cumsum(c) - c + jnp.full((_L,), car, jnp.int32)
        offs[pl.ds(t * _L, _L)] = excl
        cur[pl.ds(t * _L, _L)] = excl
        return car + jnp.sum(c)

    lax.fori_loop(0, _NB // _L, prefix, jnp.int32(0))
    offs[pl.ds(_NB, _L)] = jnp.full((_L,), jnp.int32(_N))

    def place_chunk(j, _):
        base = j * _L
        bk = bkt[pl.ds(base, _L)]
        rank = zero16i
        for d in range(1, _L):
            nb = plsc.load_gather(bkt, [jnp.maximum(lane - d, 0) + base])
            rank = rank + jnp.where((lane >= d) & (bk == nb), 1, 0)
        pos = plsc.load_gather(cur, [bk]) + rank
        plsc.addupdate_scatter(cur, [bk], ones16)
        plsc.store_scatter(xs, [pos], xv[pl.ds(base, _L)])
        plsc.store_scatter(ys, [pos], yv[pl.ds(base, _L)])
        plsc.store_scatter(zs, [pos], zv[pl.ds(base, _L)])
        plsc.store_scatter(pidx, [pos], lane + base)
        return 0

    lax.fori_loop(0, _NCHUNK, place_chunk, 0)

    def do_centre(i, gl, idxg_s):
        isplat = jnp.full((_L,), i, jnp.int32)
        cx = plsc.load_gather(ccx, [isplat])
        cy = plsc.load_gather(ccy, [isplat])
        cz = plsc.load_gather(ccz, [isplat])

        xb0v = (jnp.maximum(cx - _R, 0.0) * float(_NBX)).astype(jnp.int32)
        xb1v = (jnp.minimum(cx + _R, _SUB1) * float(_NBX)).astype(jnp.int32)
        yb0v = (jnp.maximum(cy - _R, 0.0) * float(_NBX)).astype(jnp.int32)
        yb1v = (jnp.minimum(cy + _R, _SUB1) * float(_NBX)).astype(jnp.int32)
        xb0 = jnp.sum(jnp.where(lane == 0, xb0v, 0))
        xb1 = jnp.sum(jnp.where(lane == 0, xb1v, 0))
        yb0 = jnp.sum(jnp.where(lane == 0, yb0v, 0))
        yb1 = jnp.sum(jnp.where(lane == 0, yb1v, 0))

        def strip(xb, offv_s):
            s0 = jnp.sum(jnp.where(
                lane == 0,
                plsc.load_gather(offs, [jnp.full((_L,), xb * _NBX + yb0,
                                                 jnp.int32)]), 0))
            s1 = jnp.sum(jnp.where(
                lane == 0,
                plsc.load_gather(offs, [jnp.full((_L,), xb * _NBX + yb1 + 1,
                                                 jnp.int32)]), 0))

            def chunkq(j, offv):
                base = j * _L
                pi = lane + base
                dx = xs[pl.ds(base, _L)] - cx
                dy = ys[pl.ds(base, _L)] - cy
                dz = zs[pl.ds(base, _L)] - cz
                d2 = dx * dx + dy * dy + dz * dz
                msk = (d2 < _T2) & (pi >= s0) & (pi < s1)
                pos = offv + plsc.cumsum(msk.astype(jnp.int32)) - 1
                plsc.store_scatter(cd, [pos], d2, mask=msk)
                plsc.store_scatter(ci, [pos], pidx[pl.ds(base, _L)], mask=msk)
                return offv + plsc.all_reduce_population_count(msk)

            return lax.fori_loop(s0 // _L, (s1 + _L - 1) // _L, chunkq, offv_s)

        offv = lax.fori_loop(xb0, xb1 + 1, strip, zero16i)
        mcount = jnp.sum(jnp.where(lane == 0, offv, 0))

        def no_cand(_):
            def amin_chunk(j, st):
                rmin, ridx = st
                base = j * _L
                dx = xv[pl.ds(base, _L)] - cx
                dy = yv[pl.ds(base, _L)] - cy
                dz = zv[pl.ds(base, _L)] - cz
                d2 = dx * dx + dy * dy + dz * dz
                upd = d2 < rmin
                return jnp.where(upd, d2, rmin), jnp.where(upd, lane + base, ridx)

            rmin, ridx = lax.fori_loop(0, _NCHUNK, amin_chunk, (inf16, zero16i))
            mval = jnp.min(rmin)
            am = jnp.min(jnp.where(rmin == mval, ridx, jnp.int32(_BIGI)))
            cd[pl.ds(0, _L)] = jnp.where(lane == 0, jnp.float32(0.0), inf16)
            ci[pl.ds(0, _L)] = jnp.full((_L,), am, jnp.int32)
            return jnp.int32(1)

        def have_cand(_):
            cd[pl.ds(mcount, _L)] = inf16
            return mcount

        m = lax.cond(mcount == 0, no_cand, have_cand, 0)
        nc = (m + _L - 1) // _L

        def merge(c, st):
            k0, v0, k1, v1 = st
            dk, dv = plsc.sort_key_val(cd[pl.ds(c * _L, _L)],
                                       ci[pl.ds(c * _L, _L)],
                                       descending=True)
            m1 = k1 <= dk
            tk = jnp.where(m1, k1, dk)
            tv = jnp.where(m1, v1, dv)
            tk, tv = plsc.sort_key_val(tk, tv, descending=True)
            m2 = k0 <= tk
            lok = jnp.where(m2, k0, tk)
            lov = jnp.where(m2, v0, tv)
            hik = jnp.where(m2, tk, k0)
            hiv = jnp.where(m2, tv, v0)
            k0n, v0n = plsc.sort_key_val(lok, lov)
            k1n, v1n = plsc.sort_key_val(hik, hiv)
            return k0n, v0n, k1n, v1n

        k0, v0, k1, v1 = lax.fori_loop(0, nc, merge,
                                       (inf16, zero16i, inf16, zero16i))
        c0 = jnp.sum(jnp.where(lane == 0, v0, 0))
        c0v = jnp.full((_L,), c0, jnp.int32)
        idxg_s[pl.ds(gl, _L)] = jnp.where(k0 == _INF, c0v, v0)
        idxg_s[pl.ds(gl + _L, _L)] = jnp.where(k1 == _INF, c0v, v1)

    slots = ((idxg0, gidx0, rows0, gsem0, wnp0, wix0),
             (idxg1, gidx1, rows1, gsem1, wnp1, wix1))

    def visit(sg, s, idxg_s, gidx_s, rows_s, gsem_s, wnp_s, wix_s):
        g = sg * 2 + s

        @pl.when(sg > 0)
        def _():
            pltpu.make_async_copy(idxg_s, idxo.at[pl.ds(0, _GR)], wix_s).wait()
            pltpu.make_async_copy(aug.at[gidx_s], rows_s, gsem_s).wait()
            prevbase = (b * _P + cb + (g - 2) * _G) * _S
            pltpu.async_copy(rows_s, npts.at[pl.ds(prevbase, _GR)], wnp_s)

        def centre(t, carry):
            do_centre(g * _G + t, t * _S, idxg_s)
            return carry

        lax.fori_loop(0, _G, centre, jnp.int32(0))

        boff = jnp.full((_L,), b * _N, jnp.int32)
        for t in range(_GR // _L):
            gidx_s[pl.ds(t * _L, _L)] = idxg_s[pl.ds(t * _L, _L)] + boff
        rowbase = (b * _P + cb + g * _G) * _S
        pltpu.async_copy(idxg_s, idxo.at[pl.ds(rowbase, _GR)], wix_s)

        @pl.when(sg > 0)
        def _():
            pltpu.make_async_copy(rows_s, npts.at[pl.ds(0, _GR)], wnp_s).wait()

        pltpu.async_copy(aug.at[gidx_s], rows_s, gsem_s)

    def sg_body(sg, carry):
        visit(sg, 0, *slots[0])
        visit(sg, 1, *slots[1])
        return carry

    lax.fori_loop(0, _PC // (_G * 2), sg_body, jnp.int32(0))

    ng = _PC // _G
    for s in range(2):
        idxg_s, gidx_s, rows_s, gsem_s, wnp_s, wix_s = slots[s]
        pltpu.make_async_copy(aug.at[gidx_s], rows_s, gsem_s).wait()
        rowbase = (b * _P + cb + (ng - 2 + s) * _G) * _S
        pltpu.sync_copy(rows_s, npts.at[pl.ds(rowbase, _GR)])
        pltpu.make_async_copy(idxg_s, idxo.at[pl.ds(0, _GR)], wix_s).wait()


def kernel(new_xyz, xyz, points):
    cxt = jnp.transpose(new_xyz, (0, 2, 1)).reshape(-1)
    xyzt = jnp.transpose(xyz, (0, 2, 1)).reshape(-1)
    pad = jnp.zeros((_B, _N, _DP - _D), jnp.float32)
    aug = jnp.concatenate([xyz, points, pad], axis=-1).reshape(_B * _N, _DP)

    mesh = plsc.VectorSubcoreMesh(core_axis_name="c", subcore_axis_name="s",
                                  num_cores=_NC, num_subcores=_NS)
    out_type = (
        jax.ShapeDtypeStruct((_B * _P * _S, _DP), jnp.float32),
        jax.ShapeDtypeStruct((_B * _P * _S,), jnp.int32),
    )
    scratch = [
        pltpu.VMEM((_N,), jnp.float32),
        pltpu.VMEM((_N,), jnp.float32),
        pltpu.VMEM((_N,), jnp.float32),
        pltpu.VMEM((_PC,), jnp.float32),
        pltpu.VMEM((_PC,), jnp.float32),
        pltpu.VMEM((_PC,), jnp.float32),
        pltpu.VMEM((_N + _L,), jnp.float32),
        pltpu.VMEM((_N + _L,), jnp.int32),
        pltpu.VMEM((_GR,), jnp.int32),
        pltpu.VMEM((_GR,), jnp.int32),
        pltpu.VMEM((_GR,), jnp.int32),
        pltpu.VMEM((_GR,), jnp.int32),
        pltpu.VMEM((_GR, _DP), jnp.float32),
        pltpu.VMEM((_GR, _DP), jnp.float32),
        pltpu.VMEM((_N,), jnp.int32),
        pltpu.VMEM((_N,), jnp.float32),
        pltpu.VMEM((_N,), jnp.float32),
        pltpu.VMEM((_N,), jnp.float32),
        pltpu.VMEM((_N,), jnp.int32),
        pltpu.VMEM((_NBP,), jnp.int32),
        pltpu.VMEM((_NBP,), jnp.int32),
        pltpu.SemaphoreType.DMA,
        pltpu.SemaphoreType.DMA,
        pltpu.SemaphoreType.DMA,
        pltpu.SemaphoreType.DMA,
        pltpu.SemaphoreType.DMA,
        pltpu.SemaphoreType.DMA,
    ]
    f = pl.kernel(_sc_body, out_type=out_type, mesh=mesh, scratch_types=scratch,
                  compiler_params=pltpu.CompilerParams(
                      needs_layout_passes=False, use_tc_tiling_on_sc=False))
    nppad, idxf = f(cxt, xyzt, aug)
    nppad = nppad.reshape(_B, _P, _S, _DP)
    return (nppad[..., :_D], idxf.reshape(_B, _P, _S), nppad[..., :3])
